# baseline (speedup 1.0000x reference)
import functools

import jax
import jax.numpy as jnp
import numpy as np
from jax import lax
from jax.experimental import pallas as pl
from jax.experimental.pallas import tpu as pltpu
from jax.experimental.pallas import tpu_sc as plsc

NUM_BINS = 30
EPS_CLIP = 1e-7
_F16_EPS = float(np.finfo(np.float16).eps)
_SCALE = NUM_BINS - _F16_EPS
_PAD = 32

_NC, _NS, _L = 2, 16, 16
_NW = _NC * _NS
_CH = 16384
_REG = 32
_HSZ = _L * _REG
_UNR = 8


def _sc_hist_body(p_hbm, t_hbm, out_hbm, pbuf, tbuf, hist, stage,
                  sp0, sp1, st0, st1):
    wid = lax.axis_index("c") * _NS + lax.axis_index("s")
    n = p_hbm.shape[0]
    per_w = n // _NW
    nchunk = per_w // _CH
    base = wid * per_w

    zero16 = jnp.zeros((_L,), jnp.float32)
    for k in range(_HSZ // _L):
        hist[pl.ds(k * _L, _L)] = zero16

    lane_off = lax.iota(jnp.int32, _L) * _REG
    ones = jnp.ones((_L,), jnp.float32)

    def issue(c, slot, semp, semt):
        off = base + c * _CH
        pltpu.async_copy(p_hbm.at[pl.ds(off, _CH)], pbuf.at[slot], semp)
        pltpu.async_copy(t_hbm.at[pl.ds(off, _CH)], tbuf.at[slot], semt)

    def wait(slot, semp, semt):
        pltpu.make_async_copy(
            p_hbm.at[pl.ds(0, _CH)], pbuf.at[slot], semp).wait()
        pltpu.make_async_copy(
            t_hbm.at[pl.ds(0, _CH)], tbuf.at[slot], semt).wait()

    def compute(slot):
        def body(g, _):
            b0 = g * (_UNR * _L)
            for u in range(_UNR):
                pv = pbuf[slot, pl.ds(b0 + u * _L, _L)]
                tv = tbuf[slot, pl.ds(b0 + u * _L, _L)]
                idx = (jnp.abs(pv - tv) * _SCALE).astype(jnp.int32) + lane_off
                plsc.addupdate_scatter(hist, [idx], ones)
            return 0

        lax.fori_loop(0, _CH // (_UNR * _L), body, 0)

    issue(0, 0, sp0, st0)

    def chunk_pair(gp, _):
        c0 = gp * 2
        issue(c0 + 1, 1, sp1, st1)
        wait(0, sp0, st0)
        compute(0)
        issue(jnp.minimum(c0 + 2, nchunk - 1), 0, sp0, st0)
        wait(1, sp1, st1)
        compute(1)
        return 0

    lax.fori_loop(0, nchunk // 2, chunk_pair, 0)
    wait(0, sp0, st0)

    acc_a = zero16
    acc_b = zero16
    for l in range(_L):
        acc_a = acc_a + hist[pl.ds(l * _REG, _L)]
        acc_b = acc_b + hist[pl.ds(l * _REG + _L, _L)]
    stage[pl.ds(0, _L)] = acc_a
    stage[pl.ds(_L, _L)] = acc_b
    pltpu.sync_copy(stage, out_hbm.at[wid])


_sc_hist = functools.partial(
    pl.kernel,
    out_type=jax.ShapeDtypeStruct((_NW, _PAD), jnp.float32),
    mesh=plsc.VectorSubcoreMesh(core_axis_name="c", subcore_axis_name="s"),
    compiler_params=pltpu.CompilerParams(
        use_tc_tiling_on_sc=False, needs_layout_passes=False
    ),
    scratch_types=[
        pltpu.VMEM((2, _CH), jnp.float32),
        pltpu.VMEM((2, _CH), jnp.float32),
        pltpu.VMEM((_HSZ,), jnp.float32),
        pltpu.VMEM((_PAD,), jnp.float32),
        pltpu.SemaphoreType.DMA,
        pltpu.SemaphoreType.DMA,
        pltpu.SemaphoreType.DMA,
        pltpu.SemaphoreType.DMA,
    ],
)(_sc_hist_body)


def _loss_body(num_calc, hist_ref, p_ref, t_ref, out_ref):
    counts = jnp.sum(hist_ref[...], axis=0, keepdims=True)
    nvalid = jnp.sum((counts > 0).astype(jnp.float32))
    scale = num_calc * nvalid
    recip = jnp.where(counts > 0, scale / jnp.maximum(counts, 1.0), 0.0)

    p = p_ref[...]
    t = t_ref[...]
    bm, bn = p.shape
    x = jnp.abs(p - t) * _SCALE
    idx = x.astype(jnp.int32)
    table = jnp.concatenate(
        [recip, jnp.zeros((1, 128 - _PAD), jnp.float32)], axis=1
    )
    table = jnp.broadcast_to(table, (bm, 128))
    cols = []
    for k in range(bn // 128):
        idxk = idx[:, k * 128 : (k + 1) * 128]
        cols.append(
            jnp.take_along_axis(table, idxk, axis=1, mode="promise_in_bounds")
        )
    w = jnp.concatenate(cols, axis=1)

    pc = jnp.clip(p, EPS_CLIP, 1.0 - EPS_CLIP)
    bce = -(t * jnp.log(pc) + (1.0 - t) * jnp.log(1.0 - pc))
    out_ref[...] = bce * w


def kernel(pconf, gconf):
    m, n = pconf.shape
    hist = _sc_hist(pconf.reshape(-1), gconf.reshape(-1))

    bm = 256
    grid = (m // bm,)
    blk = pl.BlockSpec((bm, n), lambda i: (i, 0))
    num_calc = np.float32(m * n)
    loss = pl.pallas_call(
        functools.partial(_loss_body, num_calc),
        grid=grid,
        in_specs=[pl.BlockSpec((_NW, _PAD), lambda i: (0, 0)), blk, blk],
        out_specs=blk,
        out_shape=jax.ShapeDtypeStruct((m, n), jnp.float32),
    )(hist, pconf, gconf)
    return loss


# speedup vs baseline: 1.0071x; 1.0071x over previous
import functools

import jax
import jax.numpy as jnp
import numpy as np
from jax import lax
from jax.experimental import pallas as pl
from jax.experimental.pallas import tpu as pltpu
from jax.experimental.pallas import tpu_sc as plsc

NUM_BINS = 30
EPS_CLIP = 1e-7
_F16_EPS = float(np.finfo(np.float16).eps)
_SCALE = NUM_BINS - _F16_EPS
_PAD = 32

_NC, _NS, _L = 2, 16, 16
_NW = _NC * _NS
_CH = 16384
_REG = 33
_HSZ = 544
_UNR = 8


def _sc_hist_body(p_hbm, t_hbm, out_hbm, pbuf, tbuf, hist, stage,
                  sp0, sp1, st0, st1):
    wid = lax.axis_index("c") * _NS + lax.axis_index("s")
    n = p_hbm.shape[0]
    per_w = n // _NW
    nchunk = per_w // _CH
    base = wid * per_w

    zero16 = jnp.zeros((_L,), jnp.float32)
    for k in range(_HSZ // _L):
        hist[pl.ds(k * _L, _L)] = zero16

    lane_off = lax.iota(jnp.int32, _L) * _REG
    ones = jnp.ones((_L,), jnp.float32)

    def issue(c, slot, semp, semt):
        off = base + c * _CH
        pltpu.async_copy(p_hbm.at[pl.ds(off, _CH)], pbuf.at[slot], semp)
        pltpu.async_copy(t_hbm.at[pl.ds(off, _CH)], tbuf.at[slot], semt)

    def wait(slot, semp, semt):
        pltpu.make_async_copy(
            p_hbm.at[pl.ds(0, _CH)], pbuf.at[slot], semp).wait()
        pltpu.make_async_copy(
            t_hbm.at[pl.ds(0, _CH)], tbuf.at[slot], semt).wait()

    def compute(slot):
        def body(g, _):
            b0 = g * (_UNR * _L)
            for u in range(_UNR):
                pv = pbuf[slot, pl.ds(b0 + u * _L, _L)]
                tv = tbuf[slot, pl.ds(b0 + u * _L, _L)]
                idx = (jnp.abs(pv - tv) * _SCALE).astype(jnp.int32) + lane_off
                plsc.addupdate_scatter(hist, [idx], ones)
            return 0

        lax.fori_loop(0, _CH // (_UNR * _L), body, 0)

    issue(0, 0, sp0, st0)

    def chunk_pair(gp, _):
        c0 = gp * 2
        issue(c0 + 1, 1, sp1, st1)
        wait(0, sp0, st0)
        compute(0)
        issue(jnp.minimum(c0 + 2, nchunk - 1), 0, sp0, st0)
        wait(1, sp1, st1)
        compute(1)
        return 0

    lax.fori_loop(0, nchunk // 2, chunk_pair, 0)
    wait(0, sp0, st0)

    acc_a = zero16
    acc_b = zero16
    for l in range(_L):
        acc_a = acc_a + hist[pl.ds(l * _REG, _L)]
        acc_b = acc_b + hist[pl.ds(l * _REG + _L, _L)]
    stage[pl.ds(0, _L)] = acc_a
    stage[pl.ds(_L, _L)] = acc_b
    pltpu.sync_copy(stage, out_hbm.at[wid])


_sc_hist = functools.partial(
    pl.kernel,
    out_type=jax.ShapeDtypeStruct((_NW, _PAD), jnp.float32),
    mesh=plsc.VectorSubcoreMesh(core_axis_name="c", subcore_axis_name="s"),
    compiler_params=pltpu.CompilerParams(
        use_tc_tiling_on_sc=False, needs_layout_passes=False
    ),
    scratch_types=[
        pltpu.VMEM((2, _CH), jnp.float32),
        pltpu.VMEM((2, _CH), jnp.float32),
        pltpu.VMEM((_HSZ,), jnp.float32),
        pltpu.VMEM((_PAD,), jnp.float32),
        pltpu.SemaphoreType.DMA,
        pltpu.SemaphoreType.DMA,
        pltpu.SemaphoreType.DMA,
        pltpu.SemaphoreType.DMA,
    ],
)(_sc_hist_body)


def _loss_body(num_calc, hist_ref, p_ref, t_ref, out_ref):
    counts = jnp.sum(hist_ref[...], axis=0, keepdims=True)
    nvalid = jnp.sum((counts > 0).astype(jnp.float32))
    scale = num_calc * nvalid
    recip = jnp.where(counts > 0, scale / jnp.maximum(counts, 1.0), 0.0)

    p = p_ref[...]
    t = t_ref[...]
    bm, bn = p.shape
    x = jnp.abs(p - t) * _SCALE
    idx = x.astype(jnp.int32)
    table = jnp.concatenate(
        [recip, jnp.zeros((1, 128 - _PAD), jnp.float32)], axis=1
    )
    table = jnp.broadcast_to(table, (bm, 128))
    cols = []
    for k in range(bn // 128):
        idxk = idx[:, k * 128 : (k + 1) * 128]
        cols.append(
            jnp.take_along_axis(table, idxk, axis=1, mode="promise_in_bounds")
        )
    w = jnp.concatenate(cols, axis=1)

    pc = jnp.clip(p, EPS_CLIP, 1.0 - EPS_CLIP)
    bce = -(t * jnp.log(pc) + (1.0 - t) * jnp.log(1.0 - pc))
    out_ref[...] = bce * w


def kernel(pconf, gconf):
    m, n = pconf.shape
    hist = _sc_hist(pconf.reshape(-1), gconf.reshape(-1))

    bm = 256
    grid = (m // bm,)
    blk = pl.BlockSpec((bm, n), lambda i: (i, 0))
    num_calc = np.float32(m * n)
    loss = pl.pallas_call(
        functools.partial(_loss_body, num_calc),
        grid=grid,
        in_specs=[pl.BlockSpec((_NW, _PAD), lambda i: (0, 0)), blk, blk],
        out_specs=blk,
        out_shape=jax.ShapeDtypeStruct((m, n), jnp.float32),
    )(hist, pconf, gconf)
    return loss
